# Initial kernel scaffold; baseline (speedup 1.0000x reference)
import jax
import jax.numpy as jnp
from jax.experimental import pallas as pl


def kernel(x, edge_index, W_self0, W_neigh0, b0, W_self1, W_neigh1, b1):
    raise NotImplementedError("write your pallas kernel here")


# speedup vs baseline: 4.2306x; 4.2306x over previous
import functools

import jax
import jax.numpy as jnp
from jax import lax
from jax.experimental import pallas as pl
from jax.experimental.pallas import tpu as pltpu
from jax.experimental.pallas import tpu_sc as plsc

N = 10000
E = 320000
IN_C = 128
HID = 256
OUT_C = 128

NC = 2
NS = 16
NW = NC * NS
K = 128
EPT = E // NW
NB = 80
VROWS = 10240
RPT = VROWS // NS
DUMMY = VROWS - 1

WA = IN_C + 16
WZ = OUT_C


def _sc_segment_sum(width):
    mesh = plsc.VectorSubcoreMesh(core_axis_name="c", subcore_axis_name="s")

    def body(table, srcp, dstp, zblk, out, src_v, dst_v, buf, acc, sem):
        c = lax.axis_index("c")
        s = lax.axis_index("s")
        g = c * NS + s
        pltpu.sync_copy(zblk, acc.at[pl.ds(s * RPT, RPT)])
        pltpu.sync_copy(srcp.at[g], src_v)
        pltpu.sync_copy(dstp.at[g], dst_v)
        plsc.subcore_barrier()

        def step(j, carry):
            pltpu.async_copy(table.at[src_v.at[j]], buf, sem).wait()
            pltpu.sync_copy(buf, acc.at[dst_v.at[j]], add=True)
            return carry

        lax.fori_loop(0, NB, step, 0)
        plsc.subcore_barrier()
        pltpu.sync_copy(acc.at[pl.ds(s * RPT, RPT)],
                        out.at[c].at[pl.ds(s * RPT, RPT)])

    return pl.kernel(
        body,
        out_type=jax.ShapeDtypeStruct((NC, VROWS, width), jnp.float32),
        mesh=mesh,
        scratch_types=[
            pltpu.VMEM((NB, K), jnp.int32),
            pltpu.VMEM((NB, K), jnp.int32),
            pltpu.VMEM((K, width), jnp.float32),
            pltpu.VMEM_SHARED((VROWS, width), jnp.float32),
            pltpu.SemaphoreType.DMA,
        ],
        compiler_params=pltpu.CompilerParams(use_tc_tiling_on_sc=False),
    )


BN = 1000


def _dense1_body(x_ref, a0_ref, a1_ref, ws0t, wn0t, b0r, ws1t, wn1t, b1r,
                 z_ref, q_ref, iv_ref):
    xb = x_ref[...]
    af = a0_ref[0] + a1_ref[0]
    onehot = (lax.broadcasted_iota(jnp.int32, (BN, WA), 1) == IN_C)
    deg = jnp.sum(jnp.where(onehot, af, 0.0), axis=1, keepdims=True)
    degc = jnp.maximum(deg, 1.0)
    m = jnp.dot(af, wn0t[...], preferred_element_type=jnp.float32)
    h0 = (jnp.dot(xb, ws0t[...], preferred_element_type=jnp.float32)
          + m / degc + b0r[...])
    t = jnp.maximum(h0, 0.0)
    s = jnp.sum(t * t, axis=1, keepdims=True) + 1.0
    inv = 1.0 / (jnp.sqrt(s) + 1e-8)
    p = t * inv
    z_ref[...] = jnp.dot(p, wn1t[...], preferred_element_type=jnp.float32)
    q_ref[...] = (jnp.dot(p, ws1t[...], preferred_element_type=jnp.float32)
                  + b1r[...])
    iv_ref[...] = jnp.broadcast_to(1.0 / degc, (BN, OUT_C))


def _dense2_body(q_ref, z0_ref, z1_ref, iv_ref, o_ref):
    o_ref[...] = q_ref[...] + (z0_ref[0] + z1_ref[0]) * iv_ref[...]


def _full(shape):
    return pl.BlockSpec(shape, lambda i: (0,) * len(shape))


def kernel(x, edge_index, W_self0, W_neigh0, b0, W_self1, W_neigh1, b1):
    dst = edge_index[0]
    src = edge_index[1]
    padw = NB * K - EPT
    srcp = jnp.concatenate(
        [src.reshape(NW, EPT),
         jnp.zeros((NW, padw), jnp.int32)], axis=1).reshape(NW, NB, K)
    dstp = jnp.concatenate(
        [dst.reshape(NW, EPT),
         jnp.full((NW, padw), DUMMY, jnp.int32)], axis=1).reshape(NW, NB, K)

    table0 = jnp.concatenate(
        [x, jnp.ones((N, 1), jnp.float32),
         jnp.zeros((N, WA - IN_C - 1), jnp.float32)], axis=1)
    zblk_a = jnp.zeros((RPT, WA), jnp.float32)
    zblk_z = jnp.zeros((RPT, WZ), jnp.float32)

    agg0 = _sc_segment_sum(WA)(table0, srcp, dstp, zblk_a)

    ws0t = W_self0.T
    wn0t = jnp.concatenate(
        [W_neigh0.T, jnp.zeros((WA - IN_C, HID), jnp.float32)], axis=0)
    ws1t = W_self1.T
    wn1t = W_neigh1.T
    b0r = b0.reshape(1, HID)
    b1r = b1.reshape(1, OUT_C)

    grid = N // BN
    z, q, iv = pl.pallas_call(
        _dense1_body,
        grid=(grid,),
        in_specs=[
            pl.BlockSpec((BN, IN_C), lambda i: (i, 0)),
            pl.BlockSpec((1, BN, WA), lambda i: (0, i, 0)),
            pl.BlockSpec((1, BN, WA), lambda i: (1, i, 0)),
            _full((IN_C, HID)),
            _full((WA, HID)),
            _full((1, HID)),
            _full((HID, OUT_C)),
            _full((HID, OUT_C)),
            _full((1, OUT_C)),
        ],
        out_specs=[
            pl.BlockSpec((BN, OUT_C), lambda i: (i, 0)),
            pl.BlockSpec((BN, OUT_C), lambda i: (i, 0)),
            pl.BlockSpec((BN, OUT_C), lambda i: (i, 0)),
        ],
        out_shape=[
            jax.ShapeDtypeStruct((N, OUT_C), jnp.float32),
            jax.ShapeDtypeStruct((N, OUT_C), jnp.float32),
            jax.ShapeDtypeStruct((N, OUT_C), jnp.float32),
        ],
    )(x, agg0, agg0, ws0t, wn0t, b0r, ws1t, wn1t, b1r)

    agg1 = _sc_segment_sum(WZ)(z, srcp, dstp, zblk_z)

    out = pl.pallas_call(
        _dense2_body,
        grid=(grid,),
        in_specs=[
            pl.BlockSpec((BN, OUT_C), lambda i: (i, 0)),
            pl.BlockSpec((1, BN, OUT_C), lambda i: (0, i, 0)),
            pl.BlockSpec((1, BN, OUT_C), lambda i: (1, i, 0)),
            pl.BlockSpec((BN, OUT_C), lambda i: (i, 0)),
        ],
        out_specs=pl.BlockSpec((BN, OUT_C), lambda i: (i, 0)),
        out_shape=jax.ShapeDtypeStruct((N, OUT_C), jnp.float32),
    )(q, agg1, agg1, iv)
    return out


# speedup vs baseline: 7.8795x; 1.8625x over previous
import functools

import jax
import jax.numpy as jnp
from jax import lax
from jax.experimental import pallas as pl
from jax.experimental.pallas import tpu as pltpu
from jax.experimental.pallas import tpu_sc as plsc

N = 10000
E = 320000
IN_C = 128
HID = 256
OUT_C = 128

NC = 2
NS = 16
NW = NC * NS
K = 128
EPT = E // NW
NB = 80
CH = 8
NH = 2
VROWS = 10112
RPT = VROWS // NS
TPT = N // NS
DUMMY = VROWS - 1

WA = IN_C + 16
HWA = WA // NH
WZ = OUT_C
HWZ = WZ // NH


def _sc_segment_sum(width):
    mesh = plsc.VectorSubcoreMesh(core_axis_name="c", subcore_axis_name="s")

    def body(table_h, srcp, dstp, zblk, out, src_v, dst_v, buf_a, buf_b,
             table_s, acc, sem_a, sem_b):
        c = lax.axis_index("c")
        s = lax.axis_index("s")
        g = c * NS + s
        bufs = (buf_a, buf_b)
        sems = (sem_a, sem_b)

        for h in range(NH):
            pltpu.sync_copy(table_h.at[h, pl.ds(s * TPT, TPT)],
                            table_s.at[pl.ds(s * TPT, TPT)])
            pltpu.sync_copy(zblk, acc.at[pl.ds(s * RPT, RPT)])
            plsc.subcore_barrier()

            def chunk(ci, carry):
                pltpu.sync_copy(srcp.at[g, pl.ds(ci * CH, CH)], src_v)
                pltpu.sync_copy(dstp.at[g, pl.ds(ci * CH, CH)], dst_v)
                handles = [pltpu.async_copy(table_s.at[src_v.at[0]], bufs[0],
                                            sems[0])]
                for k in range(CH):
                    handles[k].wait()
                    if k + 1 < CH:
                        handles.append(
                            pltpu.async_copy(table_s.at[src_v.at[k + 1]],
                                             bufs[(k + 1) % 2],
                                             sems[(k + 1) % 2]))
                    pltpu.sync_copy(bufs[k % 2], acc.at[dst_v.at[k]],
                                    add=True)
                return carry

            lax.fori_loop(0, NB // CH, chunk, 0)
            plsc.subcore_barrier()
            pltpu.sync_copy(acc.at[pl.ds(s * RPT, RPT)],
                            out.at[h].at[c].at[pl.ds(s * RPT, RPT)])

    return pl.kernel(
        body,
        out_type=jax.ShapeDtypeStruct((NH, NC, VROWS, width), jnp.float32),
        mesh=mesh,
        scratch_types=[
            pltpu.VMEM((CH, K), jnp.int32),
            pltpu.VMEM((CH, K), jnp.int32),
            pltpu.VMEM((K, width), jnp.float32),
            pltpu.VMEM((K, width), jnp.float32),
            pltpu.VMEM_SHARED((N, width), jnp.float32),
            pltpu.VMEM_SHARED((VROWS, width), jnp.float32),
            pltpu.SemaphoreType.DMA,
            pltpu.SemaphoreType.DMA,
        ],
        compiler_params=pltpu.CompilerParams(use_tc_tiling_on_sc=False),
    )


BN = 1000


def _dense1_body(x_ref, a00_ref, a01_ref, a10_ref, a11_ref, ws0t, wn0t_h0,
                 wn0t_h1, b0r, ws1t, wn1t, b1r, z_ref, q_ref, iv_ref):
    xb = x_ref[...]
    a_h0 = a00_ref[0, 0] + a01_ref[0, 0]
    a_h1 = a10_ref[0, 0] + a11_ref[0, 0]
    onehot = (lax.broadcasted_iota(jnp.int32, (BN, HWA), 1) == (IN_C - HWA))
    deg = jnp.sum(jnp.where(onehot, a_h1, 0.0), axis=1, keepdims=True)
    degc = jnp.maximum(deg, 1.0)
    m = (jnp.dot(a_h0, wn0t_h0[...], preferred_element_type=jnp.float32)
         + jnp.dot(a_h1, wn0t_h1[...], preferred_element_type=jnp.float32))
    h0 = (jnp.dot(xb, ws0t[...], preferred_element_type=jnp.float32)
          + m / degc + b0r[...])
    t = jnp.maximum(h0, 0.0)
    s = jnp.sum(t * t, axis=1, keepdims=True) + 1.0
    inv = 1.0 / (jnp.sqrt(s) + 1e-8)
    p = t * inv
    z_ref[...] = jnp.dot(p, wn1t[...], preferred_element_type=jnp.float32)
    q_ref[...] = (jnp.dot(p, ws1t[...], preferred_element_type=jnp.float32)
                  + b1r[...])
    iv_ref[...] = jnp.broadcast_to(1.0 / degc, (BN, OUT_C))


def _dense2_body(q_ref, z00_ref, z01_ref, z10_ref, z11_ref, iv_ref, o0_ref,
                 o1_ref):
    iv = iv_ref[...]
    o0_ref[...] = (q_ref[...][:, :HWZ]
                   + (z00_ref[0, 0] + z01_ref[0, 0]) * iv[:, :HWZ])
    o1_ref[...] = (q_ref[...][:, HWZ:]
                   + (z10_ref[0, 0] + z11_ref[0, 0]) * iv[:, HWZ:])


def _full(shape):
    return pl.BlockSpec(shape, lambda i: (0,) * len(shape))


def kernel(x, edge_index, W_self0, W_neigh0, b0, W_self1, W_neigh1, b1):
    dst = edge_index[0]
    src = edge_index[1]
    padw = NB * K - EPT
    srcp = jnp.concatenate(
        [src.reshape(NW, EPT),
         jnp.zeros((NW, padw), jnp.int32)], axis=1).reshape(NW, NB, K)
    dstp = jnp.concatenate(
        [dst.reshape(NW, EPT),
         jnp.full((NW, padw), DUMMY, jnp.int32)], axis=1).reshape(NW, NB, K)

    table0 = jnp.stack([
        x[:, :HWA],
        jnp.concatenate([x[:, HWA:], jnp.ones((N, 1), jnp.float32),
                         jnp.zeros((N, WA - IN_C - 1), jnp.float32)], axis=1),
    ])
    zblk_a = jnp.zeros((RPT, HWA), jnp.float32)
    zblk_z = jnp.zeros((RPT, HWZ), jnp.float32)

    agg0 = _sc_segment_sum(HWA)(table0, srcp, dstp, zblk_a)

    ws0t = W_self0.T
    wn0t = W_neigh0.T
    wn0t_h0 = wn0t[:HWA]
    wn0t_h1 = jnp.concatenate(
        [wn0t[HWA:], jnp.zeros((HWA - (IN_C - HWA), HID), jnp.float32)],
        axis=0)
    ws1t = W_self1.T
    wn1t = W_neigh1.T
    b0r = b0.reshape(1, HID)
    b1r = b1.reshape(1, OUT_C)

    grid = N // BN
    z, q, iv = pl.pallas_call(
        _dense1_body,
        grid=(grid,),
        in_specs=[
            pl.BlockSpec((BN, IN_C), lambda i: (i, 0)),
            pl.BlockSpec((1, 1, BN, HWA), lambda i: (0, 0, i, 0)),
            pl.BlockSpec((1, 1, BN, HWA), lambda i: (0, 1, i, 0)),
            pl.BlockSpec((1, 1, BN, HWA), lambda i: (1, 0, i, 0)),
            pl.BlockSpec((1, 1, BN, HWA), lambda i: (1, 1, i, 0)),
            _full((IN_C, HID)),
            _full((HWA, HID)),
            _full((HWA, HID)),
            _full((1, HID)),
            _full((HID, OUT_C)),
            _full((HID, OUT_C)),
            _full((1, OUT_C)),
        ],
        out_specs=[
            pl.BlockSpec((BN, OUT_C), lambda i: (i, 0)),
            pl.BlockSpec((BN, OUT_C), lambda i: (i, 0)),
            pl.BlockSpec((BN, OUT_C), lambda i: (i, 0)),
        ],
        out_shape=[
            jax.ShapeDtypeStruct((N, OUT_C), jnp.float32),
            jax.ShapeDtypeStruct((N, OUT_C), jnp.float32),
            jax.ShapeDtypeStruct((N, OUT_C), jnp.float32),
        ],
    )(x, agg0, agg0, agg0, agg0, ws0t, wn0t_h0, wn0t_h1, b0r, ws1t, wn1t,
      b1r)

    tablez = jnp.stack([z[:, :HWZ], z[:, HWZ:]])
    agg1 = _sc_segment_sum(HWZ)(tablez, srcp, dstp, zblk_z)

    o0, o1 = pl.pallas_call(
        _dense2_body,
        grid=(grid,),
        in_specs=[
            pl.BlockSpec((BN, OUT_C), lambda i: (i, 0)),
            pl.BlockSpec((1, 1, BN, HWZ), lambda i: (0, 0, i, 0)),
            pl.BlockSpec((1, 1, BN, HWZ), lambda i: (0, 1, i, 0)),
            pl.BlockSpec((1, 1, BN, HWZ), lambda i: (1, 0, i, 0)),
            pl.BlockSpec((1, 1, BN, HWZ), lambda i: (1, 1, i, 0)),
            pl.BlockSpec((BN, OUT_C), lambda i: (i, 0)),
        ],
        out_specs=[
            pl.BlockSpec((BN, HWZ), lambda i: (i, 0)),
            pl.BlockSpec((BN, HWZ), lambda i: (i, 0)),
        ],
        out_shape=[
            jax.ShapeDtypeStruct((N, HWZ), jnp.float32),
            jax.ShapeDtypeStruct((N, HWZ), jnp.float32),
        ],
    )(q, agg1, agg1, agg1, agg1, iv)
    return jnp.concatenate([o0, o1], axis=1)


# speedup vs baseline: 8.1372x; 1.0327x over previous
import functools

import jax
import jax.numpy as jnp
from jax import lax
from jax.experimental import pallas as pl
from jax.experimental.pallas import tpu as pltpu
from jax.experimental.pallas import tpu_sc as plsc

N = 10000
E = 320000
IN_C = 128
HID = 256
OUT_C = 128

NC = 2
NS = 16
NW = NC * NS
K = 128
EPT = E // NW
NB = 80
CH = 8
NH = 2
VROWS = 10112
RPT = VROWS // NS
TPT = N // NS
DUMMY = VROWS - 1

WA = IN_C + 16
HWA = WA // NH
WZ = OUT_C
HWZ = WZ // NH


def _sc_segment_sum(width):
    mesh = plsc.VectorSubcoreMesh(core_axis_name="c", subcore_axis_name="s")

    def body(table_h, srcp, dstp, zblk, out, src_v, dst_v, buf_a, buf_b,
             table_s, acc, sem_a, sem_b):
        c = lax.axis_index("c")
        s = lax.axis_index("s")
        g = c * NS + s
        bufs = (buf_a, buf_b)
        sems = (sem_a, sem_b)

        for h in range(NH):
            pltpu.sync_copy(table_h.at[h, pl.ds(s * TPT, TPT)],
                            table_s.at[pl.ds(s * TPT, TPT)])
            pltpu.sync_copy(zblk, acc.at[pl.ds(s * RPT, RPT)])
            plsc.subcore_barrier()

            def chunk(ci, carry):
                pltpu.sync_copy(srcp.at[g, pl.ds(ci * CH, CH)], src_v)
                pltpu.sync_copy(dstp.at[g, pl.ds(ci * CH, CH)], dst_v)
                handles = [pltpu.async_copy(table_s.at[src_v.at[0]], bufs[0],
                                            sems[0])]
                for k in range(CH):
                    handles[k].wait()
                    if k + 1 < CH:
                        handles.append(
                            pltpu.async_copy(table_s.at[src_v.at[k + 1]],
                                             bufs[(k + 1) % 2],
                                             sems[(k + 1) % 2]))
                    pltpu.sync_copy(bufs[k % 2], acc.at[dst_v.at[k]],
                                    add=True)
                return carry

            lax.fori_loop(0, NB // CH, chunk, 0)
            plsc.subcore_barrier()
            pltpu.sync_copy(acc.at[pl.ds(s * RPT, RPT)],
                            out.at[h].at[c].at[pl.ds(s * RPT, RPT)])

    return pl.kernel(
        body,
        out_type=jax.ShapeDtypeStruct((NH, NC, VROWS, width), jnp.float32),
        mesh=mesh,
        scratch_types=[
            pltpu.VMEM((CH, K), jnp.int32),
            pltpu.VMEM((CH, K), jnp.int32),
            pltpu.VMEM((K, width), jnp.float32),
            pltpu.VMEM((K, width), jnp.float32),
            pltpu.VMEM_SHARED((N, width), jnp.float32),
            pltpu.VMEM_SHARED((VROWS, width), jnp.float32),
            pltpu.SemaphoreType.DMA,
            pltpu.SemaphoreType.DMA,
        ],
        compiler_params=pltpu.CompilerParams(use_tc_tiling_on_sc=False),
    )


BN = 1000


def _dense1_body(x_ref, a00_ref, a01_ref, a10_ref, a11_ref, ws0t, wn0t_h0,
                 wn0t_h1, b0r, ws1t, wn1t, b1r, z_ref, q_ref, iv_ref):
    xb = x_ref[...]
    a_h0 = a00_ref[0, 0] + a01_ref[0, 0]
    a_h1 = a10_ref[0, 0] + a11_ref[0, 0]
    onehot = (lax.broadcasted_iota(jnp.int32, (BN, HWA), 1) == (IN_C - HWA))
    deg = jnp.sum(jnp.where(onehot, a_h1, 0.0), axis=1, keepdims=True)
    degc = jnp.maximum(deg, 1.0)
    m = (jnp.dot(a_h0, wn0t_h0[...], preferred_element_type=jnp.float32)
         + jnp.dot(a_h1, wn0t_h1[...], preferred_element_type=jnp.float32))
    h0 = (jnp.dot(xb, ws0t[...], preferred_element_type=jnp.float32)
          + m / degc + b0r[...])
    t = jnp.maximum(h0, 0.0)
    s = jnp.sum(t * t, axis=1, keepdims=True) + 1.0
    inv = 1.0 / (jnp.sqrt(s) + 1e-8)
    p = t * inv
    z = jnp.dot(p, wn1t[...], preferred_element_type=jnp.float32)
    z_ref[0] = z[:, :HWZ]
    z_ref[1] = z[:, HWZ:]
    q_ref[...] = (jnp.dot(p, ws1t[...], preferred_element_type=jnp.float32)
                  + b1r[...])
    iv_ref[...] = jnp.broadcast_to(1.0 / degc, (BN, 8))


def _dense2_body(q_ref, z00_ref, z01_ref, z10_ref, z11_ref, iv_ref, o_ref):
    iv = iv_ref[...][:, 0:1]
    agg = jnp.concatenate([z00_ref[0, 0] + z01_ref[0, 0],
                           z10_ref[0, 0] + z11_ref[0, 0]], axis=1)
    o_ref[...] = q_ref[...] + agg * iv


def _full(shape):
    return pl.BlockSpec(shape, lambda i: (0,) * len(shape))


def kernel(x, edge_index, W_self0, W_neigh0, b0, W_self1, W_neigh1, b1):
    dst = edge_index[0]
    src = edge_index[1]
    padw = NB * K - EPT
    srcp = jnp.concatenate(
        [src.reshape(NW, EPT),
         jnp.zeros((NW, padw), jnp.int32)], axis=1).reshape(NW, NB, K)
    dstp = jnp.concatenate(
        [dst.reshape(NW, EPT),
         jnp.full((NW, padw), DUMMY, jnp.int32)], axis=1).reshape(NW, NB, K)

    table0 = jnp.stack([
        x[:, :HWA],
        jnp.concatenate([x[:, HWA:], jnp.ones((N, 1), jnp.float32),
                         jnp.zeros((N, WA - IN_C - 1), jnp.float32)], axis=1),
    ])
    zblk_a = jnp.zeros((RPT, HWA), jnp.float32)
    zblk_z = jnp.zeros((RPT, HWZ), jnp.float32)

    agg0 = _sc_segment_sum(HWA)(table0, srcp, dstp, zblk_a)

    ws0t = W_self0.T
    wn0t = W_neigh0.T
    wn0t_h0 = wn0t[:HWA]
    wn0t_h1 = jnp.concatenate(
        [wn0t[HWA:], jnp.zeros((HWA - (IN_C - HWA), HID), jnp.float32)],
        axis=0)
    ws1t = W_self1.T
    wn1t = W_neigh1.T
    b0r = b0.reshape(1, HID)
    b1r = b1.reshape(1, OUT_C)

    grid = N // BN
    tablez, q, iv = pl.pallas_call(
        _dense1_body,
        grid=(grid,),
        in_specs=[
            pl.BlockSpec((BN, IN_C), lambda i: (i, 0)),
            pl.BlockSpec((1, 1, BN, HWA), lambda i: (0, 0, i, 0)),
            pl.BlockSpec((1, 1, BN, HWA), lambda i: (0, 1, i, 0)),
            pl.BlockSpec((1, 1, BN, HWA), lambda i: (1, 0, i, 0)),
            pl.BlockSpec((1, 1, BN, HWA), lambda i: (1, 1, i, 0)),
            _full((IN_C, HID)),
            _full((HWA, HID)),
            _full((HWA, HID)),
            _full((1, HID)),
            _full((HID, OUT_C)),
            _full((HID, OUT_C)),
            _full((1, OUT_C)),
        ],
        out_specs=[
            pl.BlockSpec((NH, BN, HWZ), lambda i: (0, i, 0)),
            pl.BlockSpec((BN, OUT_C), lambda i: (i, 0)),
            pl.BlockSpec((BN, 8), lambda i: (i, 0)),
        ],
        out_shape=[
            jax.ShapeDtypeStruct((NH, N, HWZ), jnp.float32),
            jax.ShapeDtypeStruct((N, OUT_C), jnp.float32),
            jax.ShapeDtypeStruct((N, 8), jnp.float32),
        ],
    )(x, agg0, agg0, agg0, agg0, ws0t, wn0t_h0, wn0t_h1, b0r, ws1t, wn1t,
      b1r)

    agg1 = _sc_segment_sum(HWZ)(tablez, srcp, dstp, zblk_z)

    out = pl.pallas_call(
        _dense2_body,
        grid=(grid,),
        in_specs=[
            pl.BlockSpec((BN, OUT_C), lambda i: (i, 0)),
            pl.BlockSpec((1, 1, BN, HWZ), lambda i: (0, 0, i, 0)),
            pl.BlockSpec((1, 1, BN, HWZ), lambda i: (0, 1, i, 0)),
            pl.BlockSpec((1, 1, BN, HWZ), lambda i: (1, 0, i, 0)),
            pl.BlockSpec((1, 1, BN, HWZ), lambda i: (1, 1, i, 0)),
            pl.BlockSpec((BN, 8), lambda i: (i, 0)),
        ],
        out_specs=pl.BlockSpec((BN, OUT_C), lambda i: (i, 0)),
        out_shape=jax.ShapeDtypeStruct((N, OUT_C), jnp.float32),
    )(q, agg1, agg1, agg1, agg1, iv)
    return out


# speedup vs baseline: 8.5800x; 1.0544x over previous
import functools

import jax
import jax.numpy as jnp
from jax import lax
from jax.experimental import pallas as pl
from jax.experimental.pallas import tpu as pltpu
from jax.experimental.pallas import tpu_sc as plsc

N = 10000
E = 320000
IN_C = 128
HID = 256
OUT_C = 128

NC = 2
NS = 16
NW = NC * NS
K = 128
EPT = E // NW
NB = 80
CH = 16
NH = 2
VROWS = 10112
RPT = VROWS // NS
TPT = N // NS
DUMMY = VROWS - 1

WA = IN_C + 16
HWA = WA // NH
WZ = OUT_C
HWZ = WZ // NH


def _sc_segment_sum(width):
    mesh = plsc.VectorSubcoreMesh(core_axis_name="c", subcore_axis_name="s")

    def body(table_h, srcp, dstp, zblk, out, src_v, dst_v, buf_a, buf_b,
             table_s, acc, sem_a, sem_b):
        c = lax.axis_index("c")
        s = lax.axis_index("s")
        g = c * NS + s
        bufs = (buf_a, buf_b)
        sems = (sem_a, sem_b)

        for h in range(NH):
            pltpu.sync_copy(table_h.at[h, pl.ds(s * TPT, TPT)],
                            table_s.at[pl.ds(s * TPT, TPT)])
            pltpu.sync_copy(zblk, acc.at[pl.ds(s * RPT, RPT)])
            plsc.subcore_barrier()

            def chunk(ci, carry):
                pltpu.sync_copy(srcp.at[g, pl.ds(ci * CH, CH)], src_v)
                pltpu.sync_copy(dstp.at[g, pl.ds(ci * CH, CH)], dst_v)
                handles = [pltpu.async_copy(table_s.at[src_v.at[0]], bufs[0],
                                            sems[0])]
                for k in range(CH):
                    handles[k].wait()
                    if k + 1 < CH:
                        handles.append(
                            pltpu.async_copy(table_s.at[src_v.at[k + 1]],
                                             bufs[(k + 1) % 2],
                                             sems[(k + 1) % 2]))
                    pltpu.sync_copy(bufs[k % 2], acc.at[dst_v.at[k]],
                                    add=True)
                return carry

            lax.fori_loop(0, NB // CH, chunk, 0)
            plsc.subcore_barrier()
            pltpu.sync_copy(acc.at[pl.ds(s * RPT, RPT)],
                            out.at[h].at[c].at[pl.ds(s * RPT, RPT)])

    return pl.kernel(
        body,
        out_type=jax.ShapeDtypeStruct((NH, NC, VROWS, width), jnp.float32),
        mesh=mesh,
        scratch_types=[
            pltpu.VMEM((CH, K), jnp.int32),
            pltpu.VMEM((CH, K), jnp.int32),
            pltpu.VMEM((K, width), jnp.float32),
            pltpu.VMEM((K, width), jnp.float32),
            pltpu.VMEM_SHARED((N, width), jnp.float32),
            pltpu.VMEM_SHARED((VROWS, width), jnp.float32),
            pltpu.SemaphoreType.DMA,
            pltpu.SemaphoreType.DMA,
        ],
        compiler_params=pltpu.CompilerParams(use_tc_tiling_on_sc=False),
    )


BN = 2000


def _dense1_body(x_ref, a00_ref, a01_ref, a10_ref, a11_ref, ws0t, wn0t_h0,
                 wn0t_h1, b0r, ws1t, wn1t, b1r, z_ref, q_ref, iv_ref):
    xb = x_ref[...]
    a_h0 = a00_ref[0, 0] + a01_ref[0, 0]
    a_h1 = a10_ref[0, 0] + a11_ref[0, 0]
    onehot = (lax.broadcasted_iota(jnp.int32, (BN, HWA), 1) == (IN_C - HWA))
    deg = jnp.sum(jnp.where(onehot, a_h1, 0.0), axis=1, keepdims=True)
    degc = jnp.maximum(deg, 1.0)
    m = (jnp.dot(a_h0, wn0t_h0[...], preferred_element_type=jnp.float32)
         + jnp.dot(a_h1, wn0t_h1[...], preferred_element_type=jnp.float32))
    h0 = (jnp.dot(xb, ws0t[...], preferred_element_type=jnp.float32)
          + m / degc + b0r[...])
    t = jnp.maximum(h0, 0.0)
    s = jnp.sum(t * t, axis=1, keepdims=True) + 1.0
    inv = 1.0 / (jnp.sqrt(s) + 1e-8)
    p = t * inv
    z = jnp.dot(p, wn1t[...], preferred_element_type=jnp.float32)
    z_ref[0] = z[:, :HWZ]
    z_ref[1] = z[:, HWZ:]
    q_ref[...] = (jnp.dot(p, ws1t[...], preferred_element_type=jnp.float32)
                  + b1r[...])
    iv_ref[...] = jnp.broadcast_to(1.0 / degc, (BN, 8))


def _dense2_body(q_ref, z00_ref, z01_ref, z10_ref, z11_ref, iv_ref, o_ref):
    iv = iv_ref[...][:, 0:1]
    agg = jnp.concatenate([z00_ref[0, 0] + z01_ref[0, 0],
                           z10_ref[0, 0] + z11_ref[0, 0]], axis=1)
    o_ref[...] = q_ref[...] + agg * iv


def _full(shape):
    return pl.BlockSpec(shape, lambda i: (0,) * len(shape))


def kernel(x, edge_index, W_self0, W_neigh0, b0, W_self1, W_neigh1, b1):
    dst = edge_index[0]
    src = edge_index[1]
    padw = NB * K - EPT
    srcp = jnp.concatenate(
        [src.reshape(NW, EPT),
         jnp.zeros((NW, padw), jnp.int32)], axis=1).reshape(NW, NB, K)
    dstp = jnp.concatenate(
        [dst.reshape(NW, EPT),
         jnp.full((NW, padw), DUMMY, jnp.int32)], axis=1).reshape(NW, NB, K)

    table0 = jnp.stack([
        x[:, :HWA],
        jnp.concatenate([x[:, HWA:], jnp.ones((N, 1), jnp.float32),
                         jnp.zeros((N, WA - IN_C - 1), jnp.float32)], axis=1),
    ])
    zblk_a = jnp.zeros((RPT, HWA), jnp.float32)
    zblk_z = jnp.zeros((RPT, HWZ), jnp.float32)

    agg0 = _sc_segment_sum(HWA)(table0, srcp, dstp, zblk_a)

    ws0t = W_self0.T
    wn0t = W_neigh0.T
    wn0t_h0 = wn0t[:HWA]
    wn0t_h1 = jnp.concatenate(
        [wn0t[HWA:], jnp.zeros((HWA - (IN_C - HWA), HID), jnp.float32)],
        axis=0)
    ws1t = W_self1.T
    wn1t = W_neigh1.T
    b0r = b0.reshape(1, HID)
    b1r = b1.reshape(1, OUT_C)

    grid = N // BN
    tablez, q, iv = pl.pallas_call(
        _dense1_body,
        grid=(grid,),
        in_specs=[
            pl.BlockSpec((BN, IN_C), lambda i: (i, 0)),
            pl.BlockSpec((1, 1, BN, HWA), lambda i: (0, 0, i, 0)),
            pl.BlockSpec((1, 1, BN, HWA), lambda i: (0, 1, i, 0)),
            pl.BlockSpec((1, 1, BN, HWA), lambda i: (1, 0, i, 0)),
            pl.BlockSpec((1, 1, BN, HWA), lambda i: (1, 1, i, 0)),
            _full((IN_C, HID)),
            _full((HWA, HID)),
            _full((HWA, HID)),
            _full((1, HID)),
            _full((HID, OUT_C)),
            _full((HID, OUT_C)),
            _full((1, OUT_C)),
        ],
        out_specs=[
            pl.BlockSpec((NH, BN, HWZ), lambda i: (0, i, 0)),
            pl.BlockSpec((BN, OUT_C), lambda i: (i, 0)),
            pl.BlockSpec((BN, 8), lambda i: (i, 0)),
        ],
        out_shape=[
            jax.ShapeDtypeStruct((NH, N, HWZ), jnp.float32),
            jax.ShapeDtypeStruct((N, OUT_C), jnp.float32),
            jax.ShapeDtypeStruct((N, 8), jnp.float32),
        ],
    )(x, agg0, agg0, agg0, agg0, ws0t, wn0t_h0, wn0t_h1, b0r, ws1t, wn1t,
      b1r)

    agg1 = _sc_segment_sum(HWZ)(tablez, srcp, dstp, zblk_z)

    out = pl.pallas_call(
        _dense2_body,
        grid=(grid,),
        in_specs=[
            pl.BlockSpec((BN, OUT_C), lambda i: (i, 0)),
            pl.BlockSpec((1, 1, BN, HWZ), lambda i: (0, 0, i, 0)),
            pl.BlockSpec((1, 1, BN, HWZ), lambda i: (0, 1, i, 0)),
            pl.BlockSpec((1, 1, BN, HWZ), lambda i: (1, 0, i, 0)),
            pl.BlockSpec((1, 1, BN, HWZ), lambda i: (1, 1, i, 0)),
            pl.BlockSpec((BN, 8), lambda i: (i, 0)),
        ],
        out_specs=pl.BlockSpec((BN, OUT_C), lambda i: (i, 0)),
        out_shape=jax.ShapeDtypeStruct((N, OUT_C), jnp.float32),
    )(q, agg1, agg1, agg1, agg1, iv)
    return out


# speedup vs baseline: 8.5884x; 1.0010x over previous
import jax
import jax.numpy as jnp
from jax import lax
from jax.experimental import pallas as pl
from jax.experimental.pallas import tpu as pltpu
from jax.experimental.pallas import tpu_sc as plsc

N = 10000
E = 320000
IN_C = 128
HID = 256
OUT_C = 128

NC = 2
NS = 16
NW = NC * NS
K = 128
EPT = E // NW
NB = 80
CH = 16
NH = 2
VROWS = 10112
RPT = VROWS // NS
TPT = N // NS
DUMMY = VROWS - 1

WA = IN_C + 16
HWA = WA // NH
WZ = OUT_C
HWZ = WZ // NH


def _sc_segment_sum(width):
    mesh = plsc.VectorSubcoreMesh(core_axis_name="c", subcore_axis_name="s")

    def body(table_h, srcp, dstp, zblk, out, src_v, dst_v, buf_a, buf_b,
             table_s, acc, sem_a, sem_b):
        c = lax.axis_index("c")
        s = lax.axis_index("s")
        g = c * NS + s
        bufs = (buf_a, buf_b)
        sems = (sem_a, sem_b)

        for h in range(NH):
            pltpu.sync_copy(table_h.at[h, pl.ds(s * TPT, TPT)],
                            table_s.at[pl.ds(s * TPT, TPT)])
            pltpu.sync_copy(zblk, acc.at[pl.ds(s * RPT, RPT)])
            plsc.subcore_barrier()

            def chunk(ci, carry):
                pltpu.sync_copy(srcp.at[g, pl.ds(ci * CH, CH)], src_v)
                pltpu.sync_copy(dstp.at[g, pl.ds(ci * CH, CH)], dst_v)
                handles = [pltpu.async_copy(table_s.at[src_v.at[0]], bufs[0],
                                            sems[0])]
                for k in range(CH):
                    handles[k].wait()
                    if k + 1 < CH:
                        handles.append(
                            pltpu.async_copy(table_s.at[src_v.at[k + 1]],
                                             bufs[(k + 1) % 2],
                                             sems[(k + 1) % 2]))
                    pltpu.sync_copy(bufs[k % 2], acc.at[dst_v.at[k]],
                                    add=True)
                return carry

            lax.fori_loop(0, NB // CH, chunk, 0)
            plsc.subcore_barrier()
            pltpu.sync_copy(acc.at[pl.ds(s * RPT, RPT)],
                            out.at[h].at[c].at[pl.ds(s * RPT, RPT)])

    return pl.kernel(
        body,
        out_type=jax.ShapeDtypeStruct((NH, NC, VROWS, width), jnp.float32),
        mesh=mesh,
        scratch_types=[
            pltpu.VMEM((CH, K), jnp.int32),
            pltpu.VMEM((CH, K), jnp.int32),
            pltpu.VMEM((K, width), jnp.float32),
            pltpu.VMEM((K, width), jnp.float32),
            pltpu.VMEM_SHARED((N, width), jnp.float32),
            pltpu.VMEM_SHARED((VROWS, width), jnp.float32),
            pltpu.SemaphoreType.DMA,
            pltpu.SemaphoreType.DMA,
        ],
        compiler_params=pltpu.CompilerParams(use_tc_tiling_on_sc=False),
    )


BN = 2000


def _dense1_body(x_ref, a00_ref, a01_ref, a10_ref, a11_ref, ws0t, wn0t_h0,
                 wn0t_h1, b0r, ws1t, wn1t, b1r, z_ref, q_ref, iv_ref):
    xb = x_ref[...]
    a_h0 = a00_ref[0, 0] + a01_ref[0, 0]
    a_h1 = a10_ref[0, 0] + a11_ref[0, 0]
    onehot = (lax.broadcasted_iota(jnp.int32, (BN, HWA), 1) == (IN_C - HWA))
    deg = jnp.sum(jnp.where(onehot, a_h1, 0.0), axis=1, keepdims=True)
    degc = jnp.maximum(deg, 1.0)
    m = (jnp.dot(a_h0, wn0t_h0[...], preferred_element_type=jnp.float32)
         + jnp.dot(a_h1, wn0t_h1[...], preferred_element_type=jnp.float32))
    h0 = (jnp.dot(xb, ws0t[...], preferred_element_type=jnp.float32)
          + m / degc + b0r[...])
    t = jnp.maximum(h0, 0.0)
    s = jnp.sum(t * t, axis=1, keepdims=True) + 1.0
    inv = 1.0 / (jnp.sqrt(s) + 1e-8)
    p = t * inv
    z = jnp.dot(p, wn1t[...], preferred_element_type=jnp.float32)
    z_ref[0] = z[:, :HWZ]
    z_ref[1] = z[:, HWZ:]
    q_ref[...] = (jnp.dot(p, ws1t[...], preferred_element_type=jnp.float32)
                  + b1r[...])
    iv_ref[...] = jnp.broadcast_to(1.0 / degc, (BN, 8))


def _dense2_body(q_ref, z00_ref, z01_ref, z10_ref, z11_ref, iv_ref, o_ref):
    iv = iv_ref[...][:, 0:1]
    agg = jnp.concatenate([z00_ref[0, 0] + z01_ref[0, 0],
                           z10_ref[0, 0] + z11_ref[0, 0]], axis=1)
    o_ref[...] = q_ref[...] + agg * iv


def _full(shape):
    return pl.BlockSpec(shape, lambda i: (0,) * len(shape))


def kernel(x, edge_index, W_self0, W_neigh0, b0, W_self1, W_neigh1, b1):
    dst = edge_index[0]
    src = edge_index[1]
    padw = NB * K - EPT
    srcp = jnp.concatenate(
        [src.reshape(NW, EPT),
         jnp.zeros((NW, padw), jnp.int32)], axis=1).reshape(NW, NB, K)
    dstp = jnp.concatenate(
        [dst.reshape(NW, EPT),
         jnp.full((NW, padw), DUMMY, jnp.int32)], axis=1).reshape(NW, NB, K)

    table0 = jnp.stack([
        x[:, :HWA],
        jnp.concatenate([x[:, HWA:], jnp.ones((N, 1), jnp.float32),
                         jnp.zeros((N, WA - IN_C - 1), jnp.float32)], axis=1),
    ])
    zblk_a = jnp.zeros((RPT, HWA), jnp.float32)
    zblk_z = jnp.zeros((RPT, HWZ), jnp.float32)

    agg0 = _sc_segment_sum(HWA)(table0, srcp, dstp, zblk_a)

    ws0t = W_self0.T
    wn0t = W_neigh0.T
    wn0t_h0 = wn0t[:HWA]
    wn0t_h1 = jnp.concatenate(
        [wn0t[HWA:], jnp.zeros((HWA - (IN_C - HWA), HID), jnp.float32)],
        axis=0)
    ws1t = W_self1.T
    wn1t = W_neigh1.T
    b0r = b0.reshape(1, HID)
    b1r = b1.reshape(1, OUT_C)

    grid = N // BN
    tablez, q, iv = pl.pallas_call(
        _dense1_body,
        grid=(grid,),
        in_specs=[
            pl.BlockSpec((BN, IN_C), lambda i: (i, 0)),
            pl.BlockSpec((1, 1, BN, HWA), lambda i: (0, 0, i, 0)),
            pl.BlockSpec((1, 1, BN, HWA), lambda i: (0, 1, i, 0)),
            pl.BlockSpec((1, 1, BN, HWA), lambda i: (1, 0, i, 0)),
            pl.BlockSpec((1, 1, BN, HWA), lambda i: (1, 1, i, 0)),
            _full((IN_C, HID)),
            _full((HWA, HID)),
            _full((HWA, HID)),
            _full((1, HID)),
            _full((HID, OUT_C)),
            _full((HID, OUT_C)),
            _full((1, OUT_C)),
        ],
        out_specs=[
            pl.BlockSpec((NH, BN, HWZ), lambda i: (0, i, 0)),
            pl.BlockSpec((BN, OUT_C), lambda i: (i, 0)),
            pl.BlockSpec((BN, 8), lambda i: (i, 0)),
        ],
        out_shape=[
            jax.ShapeDtypeStruct((NH, N, HWZ), jnp.float32),
            jax.ShapeDtypeStruct((N, OUT_C), jnp.float32),
            jax.ShapeDtypeStruct((N, 8), jnp.float32),
        ],
    )(x, agg0, agg0, agg0, agg0, ws0t, wn0t_h0, wn0t_h1, b0r, ws1t, wn1t,
      b1r)

    agg1 = _sc_segment_sum(HWZ)(tablez, srcp, dstp, zblk_z)

    out = pl.pallas_call(
        _dense2_body,
        grid=(grid,),
        in_specs=[
            pl.BlockSpec((BN, OUT_C), lambda i: (i, 0)),
            pl.BlockSpec((1, 1, BN, HWZ), lambda i: (0, 0, i, 0)),
            pl.BlockSpec((1, 1, BN, HWZ), lambda i: (0, 1, i, 0)),
            pl.BlockSpec((1, 1, BN, HWZ), lambda i: (1, 0, i, 0)),
            pl.BlockSpec((1, 1, BN, HWZ), lambda i: (1, 1, i, 0)),
            pl.BlockSpec((BN, 8), lambda i: (i, 0)),
        ],
        out_specs=pl.BlockSpec((BN, OUT_C), lambda i: (i, 0)),
        out_shape=jax.ShapeDtypeStruct((N, OUT_C), jnp.float32),
    )(q, agg1, agg1, agg1, agg1, iv)
    return out
